# baseline (speedup 1.0000x reference)
import functools
import math

import numpy as np
import jax
import jax.numpy as jnp
from jax import lax
from jax.experimental import pallas as pl
from jax.experimental.pallas import tpu as pltpu
from jax.experimental.pallas import tpu_sc as plsc

_R_CUT = 5.0
_GRID = 128
_MODES = 64
_C = 16
_NC, _NS, _L = 2, 16, 16
_NW = _NC * _NS

_H = np.float32(_R_CUT / (_GRID - 1))
_INV_H = np.float32(1.0) / _H
_INV_HEPS = np.float32(1.0 / (float(_H) + 1e-12))
_RMAX = np.float32(np.float32(_R_CUT) - 1e-12)


def _basis_env_np():
    r_grid = np.linspace(0.0, _R_CUT, _GRID).astype(np.float32)
    n = np.arange(_GRID, dtype=np.float64)[:, None] + 0.5
    k = np.arange(_MODES, dtype=np.float64)[None, :]
    B = np.cos(math.pi / _GRID * (n * k))
    s = np.ones((_MODES,))
    s[0] = 1.0 / math.sqrt(2.0)
    x = np.clip(r_grid / _R_CUT, 0.0, 1.0)
    env = 0.5 * (np.cos(math.pi * x) + 1.0)
    bt_env = (B * s).T * env[None, :]
    return bt_env.astype(np.float32)


_BT_ENV = _basis_env_np()


def _table_body(bt_ref, spec_ref, w_ref, out_ref):
    sb = jnp.dot(spec_ref[...], bt_ref[...], preferred_element_type=jnp.float32)
    out_ref[...] = jnp.dot(w_ref[...], sb, preferred_element_type=jnp.float32)


def _build_table(spec, w_mix, interpret=False):
    return pl.pallas_call(
        _table_body,
        out_shape=jax.ShapeDtypeStruct((_C, _GRID), jnp.float32),
        interpret=interpret,
    )(jnp.asarray(_BT_ENV), spec, w_mix)


def _make_interp(E, interpret=False):
    EPT = E // _NW
    CH = 1000
    assert E % _NW == 0 and EPT % CH == 0
    NCHUNK = EPT // CH
    NG = -(-CH // _L)
    PAD = NG * _L
    _UNROLL = 7 if NG % 7 == 0 else 1

    mesh = plsc.VectorSubcoreMesh(core_axis_name="c", subcore_axis_name="s",
                                  num_cores=_NC, num_subcores=_NS)

    @functools.partial(
        pl.kernel,
        out_type=jax.ShapeDtypeStruct((E, _C), jnp.float32),
        mesh=mesh,
        scratch_types=[
            pltpu.VMEM((_C * _GRID,), jnp.float32),
            pltpu.VMEM((PAD,), jnp.float32),
            pltpu.VMEM((CH, _C), jnp.float32),
        ],
        compiler_params=pltpu.CompilerParams(needs_layout_passes=False),
        interpret=interpret,
    )
    def k(tab_hbm, dist_hbm, out_hbm, tab_v, dist_v, out_v):
        wid = lax.axis_index("s") * _NC + lax.axis_index("c")
        base_w = wid * EPT
        pltpu.sync_copy(tab_hbm, tab_v)
        iota = lax.iota(jnp.int32, _L)

        def group_body(g):
            d = dist_v[pl.ds(g * _L, _L)]
            dq = jnp.minimum(jnp.maximum(d, 0.0), _RMAX)
            i0 = (dq * _INV_H).astype(jnp.int32)
            i0 = jnp.minimum(jnp.maximum(i0, 0), _GRID - 2)
            t = (d - i0.astype(jnp.float32) * _H) * _INV_HEPS
            erow = g * _L + iota
            mask = erow < CH
            for c in range(_C):
                idx0 = i0 + (c * _GRID)
                g0 = plsc.load_gather(tab_v, [idx0])
                g1 = plsc.load_gather(tab_v, [idx0 + 1])
                cc = jnp.full((_L,), c, jnp.int32)
                plsc.store_scatter(out_v, [erow, cc], g0 + t * (g1 - g0),
                                   mask=mask)

        def chunk_body(j, carry):
            base = base_w + j * CH
            pltpu.sync_copy(dist_hbm.at[pl.ds(base, CH)],
                            dist_v.at[pl.ds(0, CH)])
            plsc.parallel_loop(0, NG, 1, unroll=_UNROLL)(group_body)
            pltpu.sync_copy(out_v, out_hbm.at[pl.ds(base, CH)])
            return carry

        lax.fori_loop(0, NCHUNK, chunk_body, 0, unroll=False)

    return k


def kernel(dist, spec, W_mix):
    gt = _build_table(spec, W_mix)
    interp = _make_interp(dist.shape[0])
    return interp(gt.reshape(_C * _GRID), dist)


# speedup vs baseline: 10.3562x; 10.3562x over previous
import functools
import math

import numpy as np
import jax
import jax.numpy as jnp
from jax import lax
from jax.experimental import pallas as pl
from jax.experimental.pallas import tpu as pltpu
from jax.experimental.pallas import tpu_sc as plsc

_R_CUT = 5.0
_GRID = 128
_MODES = 64
_C = 16
_NC, _NS, _L = 2, 16, 16
_NW = _NC * _NS

_H = np.float32(_R_CUT / (_GRID - 1))
_INV_H = np.float32(1.0) / _H
_INV_HEPS = np.float32(1.0 / (float(_H) + 1e-12))
_RMAX = np.float32(np.float32(_R_CUT) - 1e-12)


def _basis_env_np():
    r_grid = np.linspace(0.0, _R_CUT, _GRID).astype(np.float32)
    n = np.arange(_GRID, dtype=np.float64)[:, None] + 0.5
    k = np.arange(_MODES, dtype=np.float64)[None, :]
    B = np.cos(math.pi / _GRID * (n * k))
    s = np.ones((_MODES,))
    s[0] = 1.0 / math.sqrt(2.0)
    x = np.clip(r_grid / _R_CUT, 0.0, 1.0)
    env = 0.5 * (np.cos(math.pi * x) + 1.0)
    bt_env = (B * s).T * env[None, :]
    return bt_env.astype(np.float32)


_BT_ENV = _basis_env_np()


def _table_body(bt_ref, spec_ref, w_ref, out_ref):
    sb = jnp.dot(spec_ref[...], bt_ref[...], preferred_element_type=jnp.float32)
    out_ref[...] = jnp.dot(w_ref[...], sb, preferred_element_type=jnp.float32)


def _build_table(spec, w_mix, interpret=False):
    return pl.pallas_call(
        _table_body,
        out_shape=jax.ShapeDtypeStruct((_C, _GRID), jnp.float32),
        interpret=interpret,
    )(jnp.asarray(_BT_ENV), spec, w_mix)


def _make_interp(E, interpret=False):
    CHE = 1280
    NCH = E // CHE
    assert E % CHE == 0
    NJ = -(-NCH // _NW)
    NGRP = CHE // _L
    _UNROLL = 8 if NGRP % 8 == 0 else 1
    assert NCH >= 2 * _NW

    mesh = plsc.VectorSubcoreMesh(core_axis_name="c", subcore_axis_name="s",
                                  num_cores=_NC, num_subcores=_NS)

    @functools.partial(
        pl.kernel,
        out_type=jax.ShapeDtypeStruct((_C, E), jnp.float32),
        mesh=mesh,
        scratch_types=[
            pltpu.VMEM((_C * _GRID,), jnp.float32),
            pltpu.VMEM((2, CHE), jnp.float32),
            pltpu.VMEM((2, _C, CHE), jnp.float32),
            pltpu.SemaphoreType.DMA((2,)),
            pltpu.SemaphoreType.DMA((2,)),
        ],
        compiler_params=pltpu.CompilerParams(needs_layout_passes=False),
        interpret=interpret,
    )
    def k(tab_hbm, dist_hbm, out_hbm, tab_v, dist_v, out_v, dsem, osem):
        wid = lax.axis_index("s") * _NC + lax.axis_index("c")
        pltpu.sync_copy(tab_hbm, tab_v)

        def dist_copy(j, b):
            cid = wid + _NW * j
            return pltpu.make_async_copy(
                dist_hbm.at[pl.ds(cid * CHE, CHE)], dist_v.at[b], dsem.at[b])

        def out_copy(j, b):
            cid = wid + _NW * j
            return pltpu.make_async_copy(
                out_v.at[b], out_hbm.at[:, pl.ds(cid * CHE, CHE)], osem.at[b])

        dist_copy(0, 0).start()

        def chunk(j, b):
            cid = wid + _NW * j

            @pl.when(cid < NCH)
            def _():
                dist_copy(j, b).wait()

                @pl.when(cid + _NW < NCH)
                def _():
                    dist_copy(j + 1, 1 - b).start()

                @pl.when(j >= 2)
                def _():
                    out_copy(j, b).wait()

                def group_body(g):
                    d = dist_v[b, pl.ds(g * _L, _L)]
                    dq = jnp.minimum(jnp.maximum(d, 0.0), _RMAX)
                    i0 = (dq * _INV_H).astype(jnp.int32)
                    i0 = jnp.minimum(jnp.maximum(i0, 0), _GRID - 2)
                    t = (d - i0.astype(jnp.float32) * _H) * _INV_HEPS
                    for c in range(_C):
                        idx0 = i0 + (c * _GRID)
                        g0 = plsc.load_gather(tab_v, [idx0])
                        g1 = plsc.load_gather(tab_v, [idx0 + 1])
                        out_v[b, c, pl.ds(g * _L, _L)] = g0 + t * (g1 - g0)

                plsc.parallel_loop(0, NGRP, 1, unroll=_UNROLL)(group_body)
                out_copy(j, b).start()

        @pl.loop(0, NJ, step=2)
        def _(j0):
            chunk(j0, 0)
            chunk(j0 + 1, 1)

        out_copy(0, 0).wait()
        out_copy(0, 1).wait()

    return k


def kernel(dist, spec, W_mix):
    gt = _build_table(spec, W_mix)
    interp = _make_interp(dist.shape[0])
    out_t = interp(gt.reshape(_C * _GRID), dist)
    return out_t.T
